# Initial kernel scaffold; baseline (speedup 1.0000x reference)
import jax
import jax.numpy as jnp
from jax.experimental import pallas as pl


def kernel(x, edge_index, Wq1, bq1, Wk1, bk1, Wv1, bv1, Ws1, bs1, Wb1, ln_g, ln_b, Wq2, bq2, Wk2, bk2, Wv2, bv2, Ws2, bs2, Wb2):
    raise NotImplementedError("write your pallas kernel here")


# speedup vs baseline: 13.6468x; 13.6468x over previous
import functools

import jax
import jax.numpy as jnp
import numpy as np
from jax import lax
from jax.experimental import pallas as pl
from jax.experimental.pallas import tpu as pltpu
from jax.experimental.pallas import tpu_sc as plsc

_N, _E, _D = 10000, 320000, 128
_C = 128
_ACC_W = 144
_DEN_COL = 128
_NS = 16
_B = 64
_EPS = _E // _NS
_NSEG = 10
_SEG = _EPS // _NSEG
_SEL = 24576
_N_LOC = 5000
_N_LOC_PAD = 5120


def _make_edge_kernel(n_loc, n_loc_pad, nseg, seg, b, sel, nc=2, ns=16,
                      interpret=False):
  rows_per_sub = n_loc_pad // ns
  cmpw = seg + 48
  ng = b // 16
  ngrp = seg // 16
  mesh = plsc.VectorSubcoreMesh(
      core_axis_name="c", subcore_axis_name="s",
      num_cores=nc, num_subcores=ns)

  def body(q_hbm, kv_hbm, src_hbm, dst_hbm,
           out_hbm, sel_src_hbm, sel_dst_hbm,
           src0, dst0, q0, kv0, lidx0, acc0,
           src1, dst1, q1, kv1, lidx1, acc1,
           tbuf, seg_src, seg_dst, cmp_src, cmp_dst,
           shared_acc,
           sem_i0, sem_g0, sem_s0, sem_i1, sem_g1, sem_s1):
    c = lax.axis_index("c")
    s = lax.axis_index("s")
    base = c * n_loc
    sent = (1 - c) * n_loc
    inv_sqrt = jnp.float32(1.0 / np.sqrt(_C))
    zero16 = jnp.zeros((16,), jnp.float32)
    zero16i = jnp.zeros((16,), jnp.int32)
    iota16 = lax.iota(jnp.int32, 16)

    b0 = dict(src=src0, dst=dst0, q=q0, kv=kv0, lidx=lidx0, acc=acc0,
              sem_i=sem_i0, sem_g=sem_g0, sem_s=sem_s0)
    b1 = dict(src=src1, dst=dst1, q=q1, kv=kv1, lidx=lidx1, acc=acc1,
              sem_i=sem_i1, sem_g=sem_g1, sem_s=sem_s1)

    def zrow(i, _):
      for j in range(_ACC_W // 16):
        acc0[i, pl.ds(j * 16, 16)] = zero16
        acc1[i, pl.ds(j * 16, 16)] = zero16
      return 0
    lax.fori_loop(0, b, zrow, 0)
    for t in range(rows_per_sub // b):
      pltpu.sync_copy(
          acc0, shared_acc.at[pl.ds(s * rows_per_sub + t * b, b)])
    plsc.subcore_barrier()

    sent16 = jnp.broadcast_to(sent, (16,)).astype(jnp.int32)

    def prefill(i, _):
      cmp_src[pl.ds(i * 16, 16)] = zero16i
      cmp_dst[pl.ds(i * 16, 16)] = sent16
      return 0

    def grp(g, off):
      d16 = seg_dst[pl.ds(g * 16, 16)]
      s16 = seg_src[pl.ds(g * 16, 16)]
      lg = d16 - base
      mk = jnp.logical_and(lg >= 0, lg < n_loc)
      pos = plsc.cumsum(mk.astype(jnp.int32))
      idx = off + pos - 1
      plsc.store_scatter(cmp_dst, [idx], d16, mask=mk)
      plsc.store_scatter(cmp_src, [idx], s16, mask=mk)
      return off + pos[15]

    def seg_body(k, total):
      ta = pl.multiple_of(total, 8)
      pltpu.sync_copy(src_hbm.at[s, k], seg_src)
      pltpu.sync_copy(dst_hbm.at[s, k], seg_dst)
      lax.fori_loop(0, cmpw // 16, prefill, 0)
      off = lax.fori_loop(0, ngrp, grp, 0)
      pltpu.sync_copy(cmp_src, sel_src_hbm.at[c, s, pl.ds(ta, cmpw)])
      pltpu.sync_copy(cmp_dst, sel_dst_hbm.at[c, s, pl.ds(ta, cmpw)])
      return total + jnp.bitwise_and(off + 7, -8)

    total = lax.fori_loop(0, nseg, seg_body, 0)
    total = pl.multiple_of(total, 8)
    lax.fori_loop(0, cmpw // 16, prefill, 0)
    pltpu.sync_copy(cmp_src, sel_src_hbm.at[c, s, pl.ds(total, cmpw)])
    pltpu.sync_copy(cmp_dst, sel_dst_hbm.at[c, s, pl.ds(total, cmpw)])
    nch = (total + (b - 1)) // b

    def issue_idx(i, bb):
      pltpu.async_copy(
          sel_src_hbm.at[c, s, pl.ds(i * b, b)], bb['src'], bb['sem_i'])
      pltpu.async_copy(
          sel_dst_hbm.at[c, s, pl.ds(i * b, b)], bb['dst'], bb['sem_i'])

    def wait_idx(bb):
      pltpu.make_async_copy(
          sel_src_hbm.at[c, s, pl.ds(0, b)], bb['src'], bb['sem_i']).wait()
      pltpu.make_async_copy(
          sel_dst_hbm.at[c, s, pl.ds(0, b)], bb['dst'], bb['sem_i']).wait()

    def issue_gather(bb):
      pltpu.async_copy(q_hbm.at[bb['dst']], bb['q'], bb['sem_g'])
      pltpu.async_copy(kv_hbm.at[bb['src']], bb['kv'], bb['sem_g'])

    def wait_gather(bb):
      pltpu.make_async_copy(q_hbm.at[bb['dst']], bb['q'], bb['sem_g']).wait()
      pltpu.make_async_copy(kv_hbm.at[bb['src']], bb['kv'], bb['sem_g']).wait()

    def localize(bb):
      for g in range(ng):
        dg = bb['dst'][pl.ds(g * 16, 16)]
        lg = dg - base
        ok = jnp.logical_and(lg >= 0, lg < n_loc)
        bb['lidx'][pl.ds(g * 16, 16)] = jnp.where(ok, lg, n_loc)

    def issue_scatter(bb):
      pltpu.async_copy(bb['acc'], shared_acc.at[bb['lidx']], bb['sem_s'],
                       add=True)

    def wait_scatter(bb):
      pltpu.make_async_copy(
          bb['acc'], shared_acc.at[bb['lidx']], bb['sem_s']).wait()

    def compute(bb):
      qb, kvb, accb = bb['q'], bb['kv'], bb['acc']
      fmt = plsc.PackFormat.INTERLEAVED

      def group(g, _):
        for e in range(16):
          row = g * 16 + e
          p = None
          for j in range(4):
            qe, qo = plsc.unpack(qb[row, pl.ds(j * 32, 32)], format=fmt)
            ke, ko = plsc.unpack(kvb[row, pl.ds(j * 32, 32)], format=fmt)
            pj = qe * ke + qo * ko
            p = pj if p is None else p + pj
          plsc.store_scatter(tbuf, [iota16, jnp.full((16,), e, jnp.int32)], p)
        alpha = tbuf[0, :]
        for r in range(1, 16):
          alpha = alpha + tbuf[r, :]
        ea = jnp.exp(alpha * inv_sqrt)
        plsc.store_scatter(
            accb, [g * 16 + iota16, jnp.full((16,), _DEN_COL, jnp.int32)], ea)
        for e in range(16):
          row = g * 16 + e
          sc = ea[e]
          for j in range(4):
            va, vb_ = plsc.unpack(
                kvb[row, pl.ds(128 + j * 32, 32)], format=fmt)
            accb[row, pl.ds(j * 32, 16)] = va * sc
            accb[row, pl.ds(j * 32 + 16, 16)] = vb_ * sc
        return 0

      lax.fori_loop(0, ng, group, 0)

    def half(i, cur, nxt):
      @pl.when(i < nch)
      def _():
        wait_gather(cur)

        @pl.when(i >= 2)
        def _():
          wait_scatter(cur)

        localize(cur)

        @pl.when(i + 2 < nch)
        def _():
          issue_idx(i + 2, cur)

        @pl.when(i + 1 < nch)
        def _():
          wait_idx(nxt)
          issue_gather(nxt)

        compute(cur)
        issue_scatter(cur)

    @pl.when(nch > 0)
    def _():
      issue_idx(0, b0)

    @pl.when(nch > 1)
    def _():
      issue_idx(1, b1)

    @pl.when(nch > 0)
    def _():
      wait_idx(b0)
      issue_gather(b0)

    def loop_body(i2, _):
      half(2 * i2, b0, b1)
      half(2 * i2 + 1, b1, b0)
      return 0

    lax.fori_loop(0, (nch + 1) // 2, loop_body, 0)

    @pl.when(nch >= 2)
    def _():
      wait_scatter(b0)
      wait_scatter(b1)

    @pl.when(nch == 1)
    def _():
      wait_scatter(b0)
    plsc.subcore_barrier()
    pltpu.sync_copy(
        shared_acc.at[pl.ds(s * rows_per_sub, rows_per_sub)],
        out_hbm.at[c, pl.ds(s * rows_per_sub, rows_per_sub)])

  dbuf = lambda: [
      pltpu.VMEM((b,), jnp.int32),
      pltpu.VMEM((b,), jnp.int32),
      pltpu.VMEM((b, 128), jnp.bfloat16),
      pltpu.VMEM((b, 256), jnp.bfloat16),
      pltpu.VMEM((b,), jnp.int32),
      pltpu.VMEM((b, _ACC_W), jnp.float32),
  ]
  return pl.kernel(
      body,
      out_type=(
          jax.ShapeDtypeStruct((nc, n_loc_pad, _ACC_W), jnp.float32),
          jax.ShapeDtypeStruct((nc, ns, sel), jnp.int32),
          jax.ShapeDtypeStruct((nc, ns, sel), jnp.int32),
      ),
      mesh=mesh,
      compiler_params=pltpu.CompilerParams(
          needs_layout_passes=False, use_tc_tiling_on_sc=False),
      scratch_types=(
          dbuf() + dbuf() + [
              pltpu.VMEM((16, 16), jnp.float32),
              pltpu.VMEM((seg,), jnp.int32),
              pltpu.VMEM((seg,), jnp.int32),
              pltpu.VMEM((cmpw,), jnp.int32),
              pltpu.VMEM((cmpw,), jnp.int32),
              pltpu.VMEM_SHARED((n_loc_pad, _ACC_W), jnp.float32),
          ] + [pltpu.SemaphoreType.DMA] * 6
      ),
      interpret=interpret,
  )


def _tc_proj(h, wcat, bcat, n, br=400, interpret=False):
  def body(h_ref, w_ref, b_ref, q_ref, kv_ref, xr_ref):
    y = jnp.dot(h_ref[:], w_ref[:], preferred_element_type=jnp.float32)
    y = y + b_ref[:]
    q_ref[:] = y[:, :128].astype(jnp.bfloat16)
    kv_ref[:] = y[:, 128:384].astype(jnp.bfloat16)
    xr_ref[:] = y[:, 384:]

  return pl.pallas_call(
      body,
      grid=(n // br,),
      in_specs=[
          pl.BlockSpec((br, 128), lambda i: (i, 0)),
          pl.BlockSpec((128, 512), lambda i: (0, 0)),
          pl.BlockSpec((1, 512), lambda i: (0, 0)),
      ],
      out_specs=[
          pl.BlockSpec((br, 128), lambda i: (i, 0)),
          pl.BlockSpec((br, 256), lambda i: (i, 0)),
          pl.BlockSpec((br, 128), lambda i: (i, 0)),
      ],
      out_shape=[
          jax.ShapeDtypeStruct((n, 128), jnp.bfloat16),
          jax.ShapeDtypeStruct((n, 256), jnp.bfloat16),
          jax.ShapeDtypeStruct((n, 128), jnp.float32),
      ],
      interpret=interpret,
  )(h, wcat, bcat)


def _combine(acc, xr, u, w):
  out_u = acc[:, :128]
  den = acc[:, 128:129]
  agg = jnp.where(den > 0.0, out_u / den, 0.0)
  logit = jnp.sum(agg * u, axis=1, keepdims=True) + jnp.sum(
      xr * w, axis=1, keepdims=True)
  beta = jax.nn.sigmoid(logit)
  return beta * xr + (1.0 - beta) * agg


def _tc_mid(acc, xr, u, w, g, bb, wcat, bcat, n, br=400, interpret=False):
  def body(acc_ref, xr_ref, u_ref, w_ref, g_ref, bb_ref, wc_ref, bc_ref,
           q_ref, kv_ref, xr2_ref):
    h = _combine(acc_ref[:], xr_ref[:], u_ref[:], w_ref[:])
    mu = jnp.mean(h, axis=1, keepdims=True)
    var = jnp.mean((h - mu) ** 2, axis=1, keepdims=True)
    h = (h - mu) * lax.rsqrt(var + 1e-5) * g_ref[:] + bb_ref[:]
    h = jnp.maximum(h, 0.0)
    y = jnp.dot(h, wc_ref[:], preferred_element_type=jnp.float32) + bc_ref[:]
    q_ref[:] = y[:, :128].astype(jnp.bfloat16)
    kv_ref[:] = y[:, 128:384].astype(jnp.bfloat16)
    xr2_ref[:] = y[:, 384:]

  return pl.pallas_call(
      body,
      grid=(n // br,),
      in_specs=[
          pl.BlockSpec((br, _ACC_W), lambda i: (i, 0)),
          pl.BlockSpec((br, 128), lambda i: (i, 0)),
          pl.BlockSpec((1, 128), lambda i: (0, 0)),
          pl.BlockSpec((1, 128), lambda i: (0, 0)),
          pl.BlockSpec((1, 128), lambda i: (0, 0)),
          pl.BlockSpec((1, 128), lambda i: (0, 0)),
          pl.BlockSpec((128, 512), lambda i: (0, 0)),
          pl.BlockSpec((1, 512), lambda i: (0, 0)),
      ],
      out_specs=[
          pl.BlockSpec((br, 128), lambda i: (i, 0)),
          pl.BlockSpec((br, 256), lambda i: (i, 0)),
          pl.BlockSpec((br, 128), lambda i: (i, 0)),
      ],
      out_shape=[
          jax.ShapeDtypeStruct((n, 128), jnp.bfloat16),
          jax.ShapeDtypeStruct((n, 256), jnp.bfloat16),
          jax.ShapeDtypeStruct((n, 128), jnp.float32),
      ],
      interpret=interpret,
  )(acc, xr, u, w, g, bb, wcat, bcat)


def _tc_final(acc, xr, u, w, n, br=1000, interpret=False):
  def body(acc_ref, xr_ref, u_ref, w_ref, o_ref):
    o_ref[:] = _combine(acc_ref[:][0], xr_ref[:], u_ref[:], w_ref[:])

  return pl.pallas_call(
      body,
      grid=(n // br,),
      in_specs=[
          pl.BlockSpec((1, br, _ACC_W), lambda i: (i // 5, i % 5, 0)),
          pl.BlockSpec((br, 128), lambda i: (i, 0)),
          pl.BlockSpec((1, 128), lambda i: (0, 0)),
          pl.BlockSpec((1, 128), lambda i: (0, 0)),
      ],
      out_specs=pl.BlockSpec((br, 128), lambda i: (i, 0)),
      out_shape=jax.ShapeDtypeStruct((n, 128), jnp.float32),
      interpret=interpret,
  )(acc, xr, u, w)


_PPERM = np.empty((128,), np.int64)
for _j in range(4):
  for _t in range(16):
    _PPERM[32 * _j + 2 * _t] = 32 * _j + _t
    _PPERM[32 * _j + 2 * _t + 1] = 32 * _j + 16 + _t


def _gate_vecs(wb):
  u = (wb[0:128, 0] + wb[256:384, 0]).reshape(1, 128)
  w = (wb[128:256, 0] - wb[256:384, 0]).reshape(1, 128)
  return u, w


@functools.lru_cache(maxsize=None)
def _edge_call():
  return _make_edge_kernel(_N_LOC, _N_LOC_PAD, _NSEG, _SEG, _B, _SEL)


def kernel(x, edge_index, Wq1, bq1, Wk1, bk1, Wv1, bv1, Ws1, bs1, Wb1,
           ln_g, ln_b, Wq2, bq2, Wk2, bk2, Wv2, bv2, Ws2, bs2, Wb2):
  src = edge_index[0].reshape(_NS, _NSEG, _SEG)
  dst = edge_index[1].reshape(_NS, _NSEG, _SEG)

  wcat1 = jnp.concatenate([Wq1, Wk1, Wv1[:, _PPERM], Ws1], axis=1)
  bcat1 = jnp.concatenate([bq1, bk1, bv1[_PPERM], bs1]).reshape(1, 512)
  wcat2 = jnp.concatenate([Wq2, Wk2, Wv2[:, _PPERM], Ws2], axis=1)
  bcat2 = jnp.concatenate([bq2, bk2, bv2[_PPERM], bs2]).reshape(1, 512)
  u1, w1 = _gate_vecs(Wb1)
  u2, w2 = _gate_vecs(Wb2)

  edge_call = _edge_call()
  q1, kv1, xr1 = _tc_proj(x, wcat1, bcat1, _N)
  acc1, _, _ = edge_call(q1, kv1, src, dst)
  acc1 = jnp.concatenate([acc1[0, :_N_LOC], acc1[1, :_N_LOC]], axis=0)
  q2, kv2, xr2 = _tc_mid(acc1, xr1, u1, w1, ln_g.reshape(1, 128),
                         ln_b.reshape(1, 128), wcat2, bcat2, _N)
  acc2, _, _ = edge_call(q2, kv2, src, dst)
  return _tc_final(acc2, xr2, u2, w2, _N)
